# Initial kernel scaffold; baseline (speedup 1.0000x reference)
import jax
import jax.numpy as jnp
from jax.experimental import pallas as pl


def kernel(indices, codewords):
    raise NotImplementedError("write your pallas kernel here")


# speedup vs baseline: 4.8051x; 4.8051x over previous
import functools

import jax
import jax.numpy as jnp
from jax import lax
from jax.experimental import pallas as pl
from jax.experimental.pallas import tpu as pltpu
from jax.experimental.pallas import tpu_sc as plsc

_B = 16384
_S = 200
_D = 32
_TOTAL = _B * _S
_NC = 2
_NS = 16
_NW = _NC * _NS
_RPW = _TOTAL // _NW
_CHUNK = 1024
_NGROUPS = _RPW // _CHUNK
_SUB = 128
_NSUB = _CHUNK // _SUB


@jax.jit
def _sc_gather(idx_flat, table):
  mesh = plsc.VectorSubcoreMesh(core_axis_name="c", subcore_axis_name="s")

  @functools.partial(
      pl.kernel,
      out_type=jax.ShapeDtypeStruct((_TOTAL, _D), jnp.float32),
      mesh=mesh,
      scratch_types=[
          pltpu.VMEM((_CHUNK,), jnp.int32),
          pltpu.VMEM((_CHUNK, _D), jnp.float32),
          pltpu.SemaphoreType.DMA,
      ],
      compiler_params=pltpu.CompilerParams(use_tc_tiling_on_sc=False),
  )
  def k(idx_hbm, table_hbm, out_hbm, idx_v, rows_v, sem):
    wid = lax.axis_index("s") * _NC + lax.axis_index("c")
    base = wid * _RPW

    def body(g, carry):
      off = base + g * _CHUNK
      pltpu.sync_copy(idx_hbm.at[pl.ds(off, _CHUNK)], idx_v)
      for j in range(_NSUB):
        pltpu.async_copy(
            table_hbm.at[idx_v.at[pl.ds(j * _SUB, _SUB)]],
            rows_v.at[pl.ds(j * _SUB, _SUB)],
            sem,
        )
      for j in range(_NSUB):
        pltpu.make_async_copy(
            table_hbm.at[idx_v.at[pl.ds(j * _SUB, _SUB)]],
            rows_v.at[pl.ds(j * _SUB, _SUB)],
            sem,
        ).wait()
      pltpu.sync_copy(rows_v, out_hbm.at[pl.ds(off, _CHUNK)])
      return carry

    lax.fori_loop(0, _NGROUPS, body, 0)

  return k(idx_flat, table)


def kernel(indices, codewords):
  idx_flat = indices.reshape(-1).astype(jnp.int32)
  out = _sc_gather(idx_flat, codewords)
  return out.reshape(_B, _S, _D)


# speedup vs baseline: 4.9542x; 1.0310x over previous
import functools

import jax
import jax.numpy as jnp
from jax import lax
from jax.experimental import pallas as pl
from jax.experimental.pallas import tpu as pltpu
from jax.experimental.pallas import tpu_sc as plsc

_B = 16384
_S = 200
_D = 32
_TOTAL = _B * _S
_NC = 2
_NS = 16
_NW = _NC * _NS
_RPW = _TOTAL // _NW
_CHUNK = 1024
_NGROUPS = _RPW // _CHUNK
_SUB = 128
_NSUB = _CHUNK // _SUB
_NBUF = 2


@jax.jit
def _sc_gather(idx_flat, table):
  mesh = plsc.VectorSubcoreMesh(core_axis_name="c", subcore_axis_name="s")

  @functools.partial(
      pl.kernel,
      out_type=jax.ShapeDtypeStruct((_TOTAL, _D), jnp.float32),
      mesh=mesh,
      scratch_types=[
          pltpu.VMEM((_NBUF, _CHUNK), jnp.int32),
          pltpu.VMEM((_NBUF, _CHUNK, _D), jnp.float32),
          pltpu.SemaphoreType.DMA,
          pltpu.SemaphoreType.DMA,
          pltpu.SemaphoreType.DMA,
          pltpu.SemaphoreType.DMA,
      ],
      compiler_params=pltpu.CompilerParams(use_tc_tiling_on_sc=False),
  )
  def k(idx_hbm, table_hbm, out_hbm, idx_v, rows_v, g0, g1, w0, w1):
    gsems = (g0, g1)
    wsems = (w0, w1)
    wid = lax.axis_index("s") * _NC + lax.axis_index("c")
    base = wid * _RPW

    def fire_gathers(g, b):
      off = base + g * _CHUNK
      pltpu.sync_copy(idx_hbm.at[pl.ds(off, _CHUNK)], idx_v.at[b])
      for j in range(_NSUB):
        pltpu.async_copy(
            table_hbm.at[idx_v.at[b, pl.ds(j * _SUB, _SUB)]],
            rows_v.at[b, pl.ds(j * _SUB, _SUB)],
            gsems[b],
        )

    def drain_gathers(b):
      for j in range(_NSUB):
        pltpu.make_async_copy(
            table_hbm.at[idx_v.at[b, pl.ds(j * _SUB, _SUB)]],
            rows_v.at[b, pl.ds(j * _SUB, _SUB)],
            gsems[b],
        ).wait()

    def fire_wb(g, b):
      off = base + g * _CHUNK
      pltpu.async_copy(rows_v.at[b], out_hbm.at[pl.ds(off, _CHUNK)], wsems[b])

    def wait_wb(g, b):
      off = base + g * _CHUNK
      pltpu.make_async_copy(
          rows_v.at[b], out_hbm.at[pl.ds(off, _CHUNK)], wsems[b]
      ).wait()

    for b in range(_NBUF):
      fire_gathers(b, b)

    def outer(i, carry):
      g0_ = i * _NBUF
      for b in range(_NBUF):
        g = g0_ + b
        drain_gathers(b)
        fire_wb(g, b)
        wait_wb(g, b)
        fire_gathers(g + _NBUF, b)
      return carry

    lax.fori_loop(0, _NGROUPS // _NBUF - 1, outer, 0)

    gl = _NGROUPS - _NBUF
    for b in range(_NBUF):
      drain_gathers(b)
      fire_wb(gl + b, b)
    for b in range(_NBUF):
      wait_wb(gl + b, b)

  return k(idx_flat, table)


def kernel(indices, codewords):
  idx_flat = indices.reshape(-1).astype(jnp.int32)
  out = _sc_gather(idx_flat, codewords)
  return out.reshape(_B, _S, _D)
